# baseline (speedup 1.0000x reference)
import functools

import jax
import jax.numpy as jnp
from jax import lax
from jax.experimental import pallas as pl
from jax.experimental.pallas import tpu as pltpu
from jax.experimental.pallas import tpu_sc as plsc

N = 10000
E = 320000
H = 128
C = 40

NC = 2
NS = 16
LANES = 16

CHUNK = 128
NBUF = 4
SDEPTH = 1
LOOKAHEAD = NBUF - SDEPTH
NCHUNK = 157
NLOOP = -(-NCHUNK // NBUF)
PADE = NS * NCHUNK * CHUNK
HALF = 5120
ACC_ROWS = 5232
ZPT = ACC_ROWS // NS
WPT = HALF // NS
OUT_N = NC * HALF

BLK = 1000


def _sc_scatter(hW, pk_idx):
    mesh = plsc.VectorSubcoreMesh(core_axis_name="c", subcore_axis_name="s")

    @functools.partial(
        pl.kernel,
        out_type=jax.ShapeDtypeStruct((OUT_N, H), jnp.float32),
        mesh=mesh,
        scratch_types=[
            pltpu.VMEM((NCHUNK, CHUNK), jnp.int32),
            pltpu.VMEM((NBUF, CHUNK), jnp.int32),
            pltpu.VMEM((NBUF, CHUNK), jnp.int32),
            [pltpu.VMEM((CHUNK, H), jnp.float32) for _ in range(NBUF)],
            pltpu.VMEM_SHARED((ACC_ROWS, H), jnp.float32),
            [pltpu.SemaphoreType.DMA for _ in range(NBUF)],
            [pltpu.SemaphoreType.DMA for _ in range(NBUF)],
        ],
    )
    def k(hW_hbm, pk_hbm, out_hbm, pk_v, src_r, dst_r, bufs, acc, gsems, ssems):
        c = lax.axis_index("c")
        s = lax.axis_index("s")
        buf0 = bufs[0]

        def unpack(ch, slot):
            for kk in range(CHUNK // LANES):
                v = pk_v[ch, pl.ds(kk * LANES, LANES)]
                src_r[slot, pl.ds(kk * LANES, LANES)] = v & 0xFFFF
                dst_r[slot, pl.ds(kk * LANES, LANES)] = lax.shift_right_logical(v, 16)

        def zrow(r, carry):
            for kk in range(H // LANES):
                buf0[r, pl.ds(kk * LANES, LANES)] = jnp.zeros((LANES,), jnp.float32)
            return carry

        lax.fori_loop(0, CHUNK, zrow, 0)
        zbase = s * ZPT
        done = 0
        while done < ZPT:
            n = min(CHUNK, ZPT - done)
            pltpu.sync_copy(buf0.at[pl.ds(0, n)], acc.at[pl.ds(zbase + done, n)])
            done += n

        pltpu.sync_copy(pk_hbm.at[c, s], pk_v)
        for ch in range(LOOKAHEAD):
            unpack(ch, ch)
            pltpu.async_copy(hW_hbm.at[src_r.at[ch]], bufs[ch], gsems[ch])

        plsc.subcore_barrier()

        def step(i, carry):
            for b in range(NBUF):
                ch = NBUF * i + b

                @pl.when(ch < NCHUNK)
                def _(b=b, ch=ch):
                    buf = bufs[b]
                    pltpu.make_async_copy(hW_hbm.at[src_r.at[b]], buf, gsems[b]).wait()
                    pltpu.async_copy(buf, acc.at[dst_r.at[b]], ssems[b], add=True)

                    nxt = ch + LOOKAHEAD
                    nb = (b + LOOKAHEAD) % NBUF

                    @pl.when(nxt < NCHUNK)
                    def _():
                        @pl.when(ch >= SDEPTH)
                        def _():
                            pltpu.make_async_copy(
                                hW_hbm.at[src_r.at[0]], bufs[nb], ssems[nb]).wait()

                        unpack(nxt, nb)
                        pltpu.async_copy(hW_hbm.at[src_r.at[nb]], bufs[nb], gsems[nb])
            return carry

        lax.fori_loop(0, NLOOP, step, 0)
        for b in range(NBUF):
            pltpu.make_async_copy(hW_hbm.at[src_r.at[0]], bufs[b], ssems[b]).wait()

        plsc.subcore_barrier()
        lbase = s * WPT
        obase = c * HALF + lbase
        off = 0
        while off < WPT:
            nrows = min(CHUNK, WPT - off)
            pltpu.sync_copy(acc.at[pl.ds(lbase + off, nrows)], buf0.at[pl.ds(0, nrows)])
            pltpu.sync_copy(buf0.at[pl.ds(0, nrows)], out_hbm.at[pl.ds(obase + off, nrows)])
            off += nrows

    return k(hW, pk_idx)


def _mm_kernel(x_ref, w_ref, b_ref, o_ref):
    o_ref[...] = (
        jnp.dot(x_ref[...], w_ref[...], preferred_element_type=jnp.float32) + b_ref[...]
    )


def _mm(x, WT, b):
    K = WT.shape[1]
    return pl.pallas_call(
        _mm_kernel,
        grid=(N // BLK,),
        in_specs=[
            pl.BlockSpec((BLK, H), lambda i: (i, 0)),
            pl.BlockSpec((H, K), lambda i: (0, 0)),
            pl.BlockSpec((1, K), lambda i: (0, 0)),
        ],
        out_specs=pl.BlockSpec((BLK, K), lambda i: (i, 0)),
        out_shape=jax.ShapeDtypeStruct((N, K), jnp.float32),
    )(x, WT, b.reshape(1, K))


def _gru_core(a, h, wih, bih, whh, bhh):
    gi = jnp.dot(a, wih, preferred_element_type=jnp.float32) + bih
    gh = jnp.dot(h, whh, preferred_element_type=jnp.float32) + bhh
    r = jax.nn.sigmoid(gi[:, :H] + gh[:, :H])
    z = jax.nn.sigmoid(gi[:, H : 2 * H] + gh[:, H : 2 * H])
    n = jnp.tanh(gi[:, 2 * H :] + r * gh[:, 2 * H :])
    return jnp.maximum((1.0 - z) * n + z * h, 0.0)


def _gru_kernel(a_ref, h_ref, wih_ref, bih_ref, whh_ref, bhh_ref, o_ref):
    o_ref[...] = _gru_core(a_ref[...], h_ref[...], wih_ref[...], bih_ref[...],
                           whh_ref[...], bhh_ref[...])


def _gru_mm_kernel(a_ref, h_ref, wih_ref, bih_ref, whh_ref, bhh_ref,
                   wn_ref, bn_ref, o_ref, m_ref):
    hn = _gru_core(a_ref[...], h_ref[...], wih_ref[...], bih_ref[...],
                   whh_ref[...], bhh_ref[...])
    o_ref[...] = hn
    m_ref[...] = jnp.dot(hn, wn_ref[...], preferred_element_type=jnp.float32) + bn_ref[...]


def _gru_mm(part, h, WihT, bih, WhhT, bhh, WnT, bn):
    return pl.pallas_call(
        _gru_mm_kernel,
        grid=(N // BLK,),
        in_specs=[
            pl.BlockSpec((BLK, H), lambda i: (i, 0)),
            pl.BlockSpec((BLK, H), lambda i: (i, 0)),
            pl.BlockSpec((H, 3 * H), lambda i: (0, 0)),
            pl.BlockSpec((1, 3 * H), lambda i: (0, 0)),
            pl.BlockSpec((H, 3 * H), lambda i: (0, 0)),
            pl.BlockSpec((1, 3 * H), lambda i: (0, 0)),
            pl.BlockSpec((H, H), lambda i: (0, 0)),
            pl.BlockSpec((1, H), lambda i: (0, 0)),
        ],
        out_specs=[
            pl.BlockSpec((BLK, H), lambda i: (i, 0)),
            pl.BlockSpec((BLK, H), lambda i: (i, 0)),
        ],
        out_shape=[
            jax.ShapeDtypeStruct((N, H), jnp.float32),
            jax.ShapeDtypeStruct((N, H), jnp.float32),
        ],
    )(part, h, WihT, bih.reshape(1, 3 * H), WhhT, bhh.reshape(1, 3 * H),
      WnT, bn.reshape(1, H))


def _gru(part, h, WihT, bih, WhhT, bhh):
    return pl.pallas_call(
        _gru_kernel,
        grid=(N // BLK,),
        in_specs=[
            pl.BlockSpec((BLK, H), lambda i: (i, 0)),
            pl.BlockSpec((BLK, H), lambda i: (i, 0)),
            pl.BlockSpec((H, 3 * H), lambda i: (0, 0)),
            pl.BlockSpec((1, 3 * H), lambda i: (0, 0)),
            pl.BlockSpec((H, 3 * H), lambda i: (0, 0)),
            pl.BlockSpec((1, 3 * H), lambda i: (0, 0)),
        ],
        out_specs=pl.BlockSpec((BLK, H), lambda i: (i, 0)),
        out_shape=jax.ShapeDtypeStruct((N, H), jnp.float32),
    )(part, h, WihT, bih.reshape(1, 3 * H), WhhT, bhh.reshape(1, 3 * H))


def _head_kernel(h_ref, w1_ref, b1_ref, w2_ref, b2_ref, o_ref, acc_ref):
    i = pl.program_id(0)

    @pl.when(i == 0)
    def _():
        acc_ref[...] = jnp.zeros_like(acc_ref)

    blk = h_ref[...]
    acc_ref[...] += jnp.sum(blk.reshape(BLK // 8, 8, H), axis=0)

    @pl.when(i == N // BLK - 1)
    def _():
        m = jnp.sum(acc_ref[...], axis=0, keepdims=True) * (1.0 / N)
        o1 = jnp.dot(m, w1_ref[...], preferred_element_type=jnp.float32) + b1_ref[...]
        o1 = jnp.where(o1 > 0.0, o1, jnp.exp(o1) - 1.0)
        o2 = jnp.dot(o1, w2_ref[...], preferred_element_type=jnp.float32) + b2_ref[...]
        mx = jnp.max(o2, axis=0, keepdims=True)
        lse = mx + jnp.log(jnp.sum(jnp.exp(o2 - mx), axis=0, keepdims=True))
        o_ref[...] = o2 - lse


def _head(h, fc1_WT, fc1_b, fc2_WT, fc2_b):
    return pl.pallas_call(
        _head_kernel,
        grid=(N // BLK,),
        in_specs=[
            pl.BlockSpec((BLK, H), lambda i: (i, 0)),
            pl.BlockSpec((H, H), lambda i: (0, 0)),
            pl.BlockSpec((1, H), lambda i: (0, 0)),
            pl.BlockSpec((H, C), lambda i: (0, 0)),
            pl.BlockSpec((1, C), lambda i: (0, 0)),
        ],
        out_specs=pl.BlockSpec((1, C), lambda i: (0, 0)),
        out_shape=jax.ShapeDtypeStruct((1, C), jnp.float32),
        scratch_shapes=[pltpu.VMEM((8, H), jnp.float32)],
    )(h, fc1_WT, fc1_b.reshape(1, H), fc2_WT, fc2_b.reshape(1, C))


def kernel(h, edge_index, e, l0_W, l0_b, l0_Wih, l0_bih, l0_Whh, l0_bhh,
           l1_W, l1_b, l1_Wih, l1_bih, l1_Whh, l1_bhh,
           l2_W, l2_b, l2_Wih, l2_bih, l2_Whh, l2_bhh,
           l3_W, l3_b, l3_Wih, l3_bih, l3_Whh, l3_bhh,
           fc1_W, fc1_b, fc2_W, fc2_b):
    src = edge_index[0]
    dst = edge_index[1]
    pad = PADE - E
    srcp = jnp.concatenate([src, jnp.zeros((pad,), jnp.int32)])
    pcore = []
    for c in range(NC):
        local = dst - c * HALF
        local = jnp.where((local >= 0) & (local < HALF), local, HALF)
        localp = jnp.concatenate([local, jnp.full((pad,), HALF, jnp.int32)])
        pcore.append(srcp | (localp << 16))
    pkp = jnp.stack(pcore).reshape(NC, NS, NCHUNK, CHUNK)

    layers = [
        (l0_W, l0_b, l0_Wih, l0_bih, l0_Whh, l0_bhh),
        (l1_W, l1_b, l1_Wih, l1_bih, l1_Whh, l1_bhh),
        (l2_W, l2_b, l2_Wih, l2_bih, l2_Whh, l2_bhh),
        (l3_W, l3_b, l3_Wih, l3_bih, l3_Whh, l3_bhh),
    ]
    x = h
    hW = _mm(x, l0_W.T, l0_b)
    for l, (W, b, Wih, bih, Whh, bhh) in enumerate(layers):
        part = _sc_scatter(hW, pkp)
        if l < 3:
            Wn, bn = layers[l + 1][0], layers[l + 1][1]
            x, hW = _gru_mm(part, x, Wih.T, bih, Whh.T, bhh, Wn.T, bn)
        else:
            x = _gru(part, x, Wih.T, bih, Whh.T, bhh)
    return _head(x, fc1_W.T, fc1_b, fc2_W.T, fc2_b)


# speedup vs baseline: 1.0009x; 1.0009x over previous
import functools

import jax
import jax.numpy as jnp
from jax import lax
from jax.experimental import pallas as pl
from jax.experimental.pallas import tpu as pltpu
from jax.experimental.pallas import tpu_sc as plsc

N = 10000
E = 320000
H = 128
C = 40

NC = 2
NS = 16
LANES = 16

CHUNK = 128
NBUF = 4
SDEPTH = 2
LOOKAHEAD = NBUF - SDEPTH
NCHUNK = 157
NLOOP = -(-NCHUNK // NBUF)
PADE = NS * NCHUNK * CHUNK
HALF = 5120
ACC_ROWS = 5232
ZPT = ACC_ROWS // NS
WPT = HALF // NS
OUT_N = NC * HALF

BLK = 1000


def _sc_scatter(hW, pk_idx):
    mesh = plsc.VectorSubcoreMesh(core_axis_name="c", subcore_axis_name="s")

    @functools.partial(
        pl.kernel,
        out_type=jax.ShapeDtypeStruct((OUT_N, H), jnp.float32),
        mesh=mesh,
        scratch_types=[
            pltpu.VMEM((NCHUNK, CHUNK), jnp.int32),
            pltpu.VMEM((NBUF, CHUNK), jnp.int32),
            pltpu.VMEM((NBUF, CHUNK), jnp.int32),
            [pltpu.VMEM((CHUNK, H), jnp.float32) for _ in range(NBUF)],
            pltpu.VMEM_SHARED((ACC_ROWS, H), jnp.float32),
            [pltpu.SemaphoreType.DMA for _ in range(NBUF)],
            [pltpu.SemaphoreType.DMA for _ in range(NBUF)],
        ],
    )
    def k(hW_hbm, pk_hbm, out_hbm, pk_v, src_r, dst_r, bufs, acc, gsems, ssems):
        c = lax.axis_index("c")
        s = lax.axis_index("s")
        buf0 = bufs[0]

        def unpack(ch, slot):
            for kk in range(CHUNK // LANES):
                v = pk_v[ch, pl.ds(kk * LANES, LANES)]
                src_r[slot, pl.ds(kk * LANES, LANES)] = v & 0xFFFF
                dst_r[slot, pl.ds(kk * LANES, LANES)] = lax.shift_right_logical(v, 16)

        def zrow(r, carry):
            for kk in range(H // LANES):
                buf0[r, pl.ds(kk * LANES, LANES)] = jnp.zeros((LANES,), jnp.float32)
            return carry

        lax.fori_loop(0, CHUNK, zrow, 0)
        zbase = s * ZPT
        done = 0
        while done < ZPT:
            n = min(CHUNK, ZPT - done)
            pltpu.sync_copy(buf0.at[pl.ds(0, n)], acc.at[pl.ds(zbase + done, n)])
            done += n

        pltpu.sync_copy(pk_hbm.at[c, s], pk_v)
        for ch in range(LOOKAHEAD):
            unpack(ch, ch)
            pltpu.async_copy(hW_hbm.at[src_r.at[ch]], bufs[ch], gsems[ch])

        plsc.subcore_barrier()

        def step(i, carry):
            for b in range(NBUF):
                ch = NBUF * i + b

                @pl.when(ch < NCHUNK)
                def _(b=b, ch=ch):
                    buf = bufs[b]
                    pltpu.make_async_copy(hW_hbm.at[src_r.at[b]], buf, gsems[b]).wait()
                    pltpu.async_copy(buf, acc.at[dst_r.at[b]], ssems[b], add=True)

                    nxt = ch + LOOKAHEAD
                    nb = (b + LOOKAHEAD) % NBUF

                    @pl.when(nxt < NCHUNK)
                    def _():
                        @pl.when(ch >= SDEPTH)
                        def _():
                            pltpu.make_async_copy(
                                hW_hbm.at[src_r.at[0]], bufs[nb], ssems[nb]).wait()

                        unpack(nxt, nb)
                        pltpu.async_copy(hW_hbm.at[src_r.at[nb]], bufs[nb], gsems[nb])
            return carry

        lax.fori_loop(0, NLOOP, step, 0)
        for b in range(NBUF):
            pltpu.make_async_copy(hW_hbm.at[src_r.at[0]], bufs[b], ssems[b]).wait()

        plsc.subcore_barrier()
        lbase = s * WPT
        obase = c * HALF + lbase
        off = 0
        while off < WPT:
            nrows = min(CHUNK, WPT - off)
            pltpu.sync_copy(acc.at[pl.ds(lbase + off, nrows)], buf0.at[pl.ds(0, nrows)])
            pltpu.sync_copy(buf0.at[pl.ds(0, nrows)], out_hbm.at[pl.ds(obase + off, nrows)])
            off += nrows

    return k(hW, pk_idx)


def _mm_kernel(x_ref, w_ref, b_ref, o_ref):
    o_ref[...] = (
        jnp.dot(x_ref[...], w_ref[...], preferred_element_type=jnp.float32) + b_ref[...]
    )


def _mm(x, WT, b):
    K = WT.shape[1]
    return pl.pallas_call(
        _mm_kernel,
        grid=(N // BLK,),
        in_specs=[
            pl.BlockSpec((BLK, H), lambda i: (i, 0)),
            pl.BlockSpec((H, K), lambda i: (0, 0)),
            pl.BlockSpec((1, K), lambda i: (0, 0)),
        ],
        out_specs=pl.BlockSpec((BLK, K), lambda i: (i, 0)),
        out_shape=jax.ShapeDtypeStruct((N, K), jnp.float32),
    )(x, WT, b.reshape(1, K))


def _gru_core(a, h, wih, bih, whh, bhh):
    gi = jnp.dot(a, wih, preferred_element_type=jnp.float32) + bih
    gh = jnp.dot(h, whh, preferred_element_type=jnp.float32) + bhh
    r = jax.nn.sigmoid(gi[:, :H] + gh[:, :H])
    z = jax.nn.sigmoid(gi[:, H : 2 * H] + gh[:, H : 2 * H])
    n = jnp.tanh(gi[:, 2 * H :] + r * gh[:, 2 * H :])
    return jnp.maximum((1.0 - z) * n + z * h, 0.0)


def _gru_kernel(a_ref, h_ref, wih_ref, bih_ref, whh_ref, bhh_ref, o_ref):
    o_ref[...] = _gru_core(a_ref[...], h_ref[...], wih_ref[...], bih_ref[...],
                           whh_ref[...], bhh_ref[...])


def _gru_mm_kernel(a_ref, h_ref, wih_ref, bih_ref, whh_ref, bhh_ref,
                   wn_ref, bn_ref, o_ref, m_ref):
    hn = _gru_core(a_ref[...], h_ref[...], wih_ref[...], bih_ref[...],
                   whh_ref[...], bhh_ref[...])
    o_ref[...] = hn
    m_ref[...] = jnp.dot(hn, wn_ref[...], preferred_element_type=jnp.float32) + bn_ref[...]


def _gru_mm(part, h, WihT, bih, WhhT, bhh, WnT, bn):
    return pl.pallas_call(
        _gru_mm_kernel,
        grid=(N // BLK,),
        in_specs=[
            pl.BlockSpec((BLK, H), lambda i: (i, 0)),
            pl.BlockSpec((BLK, H), lambda i: (i, 0)),
            pl.BlockSpec((H, 3 * H), lambda i: (0, 0)),
            pl.BlockSpec((1, 3 * H), lambda i: (0, 0)),
            pl.BlockSpec((H, 3 * H), lambda i: (0, 0)),
            pl.BlockSpec((1, 3 * H), lambda i: (0, 0)),
            pl.BlockSpec((H, H), lambda i: (0, 0)),
            pl.BlockSpec((1, H), lambda i: (0, 0)),
        ],
        out_specs=[
            pl.BlockSpec((BLK, H), lambda i: (i, 0)),
            pl.BlockSpec((BLK, H), lambda i: (i, 0)),
        ],
        out_shape=[
            jax.ShapeDtypeStruct((N, H), jnp.float32),
            jax.ShapeDtypeStruct((N, H), jnp.float32),
        ],
    )(part, h, WihT, bih.reshape(1, 3 * H), WhhT, bhh.reshape(1, 3 * H),
      WnT, bn.reshape(1, H))


def _gru(part, h, WihT, bih, WhhT, bhh):
    return pl.pallas_call(
        _gru_kernel,
        grid=(N // BLK,),
        in_specs=[
            pl.BlockSpec((BLK, H), lambda i: (i, 0)),
            pl.BlockSpec((BLK, H), lambda i: (i, 0)),
            pl.BlockSpec((H, 3 * H), lambda i: (0, 0)),
            pl.BlockSpec((1, 3 * H), lambda i: (0, 0)),
            pl.BlockSpec((H, 3 * H), lambda i: (0, 0)),
            pl.BlockSpec((1, 3 * H), lambda i: (0, 0)),
        ],
        out_specs=pl.BlockSpec((BLK, H), lambda i: (i, 0)),
        out_shape=jax.ShapeDtypeStruct((N, H), jnp.float32),
    )(part, h, WihT, bih.reshape(1, 3 * H), WhhT, bhh.reshape(1, 3 * H))


def _head_kernel(h_ref, w1_ref, b1_ref, w2_ref, b2_ref, o_ref, acc_ref):
    i = pl.program_id(0)

    @pl.when(i == 0)
    def _():
        acc_ref[...] = jnp.zeros_like(acc_ref)

    blk = h_ref[...]
    acc_ref[...] += jnp.sum(blk.reshape(BLK // 8, 8, H), axis=0)

    @pl.when(i == N // BLK - 1)
    def _():
        m = jnp.sum(acc_ref[...], axis=0, keepdims=True) * (1.0 / N)
        o1 = jnp.dot(m, w1_ref[...], preferred_element_type=jnp.float32) + b1_ref[...]
        o1 = jnp.where(o1 > 0.0, o1, jnp.exp(o1) - 1.0)
        o2 = jnp.dot(o1, w2_ref[...], preferred_element_type=jnp.float32) + b2_ref[...]
        mx = jnp.max(o2, axis=0, keepdims=True)
        lse = mx + jnp.log(jnp.sum(jnp.exp(o2 - mx), axis=0, keepdims=True))
        o_ref[...] = o2 - lse


def _head(h, fc1_WT, fc1_b, fc2_WT, fc2_b):
    return pl.pallas_call(
        _head_kernel,
        grid=(N // BLK,),
        in_specs=[
            pl.BlockSpec((BLK, H), lambda i: (i, 0)),
            pl.BlockSpec((H, H), lambda i: (0, 0)),
            pl.BlockSpec((1, H), lambda i: (0, 0)),
            pl.BlockSpec((H, C), lambda i: (0, 0)),
            pl.BlockSpec((1, C), lambda i: (0, 0)),
        ],
        out_specs=pl.BlockSpec((1, C), lambda i: (0, 0)),
        out_shape=jax.ShapeDtypeStruct((1, C), jnp.float32),
        scratch_shapes=[pltpu.VMEM((8, H), jnp.float32)],
    )(h, fc1_WT, fc1_b.reshape(1, H), fc2_WT, fc2_b.reshape(1, C))


def kernel(h, edge_index, e, l0_W, l0_b, l0_Wih, l0_bih, l0_Whh, l0_bhh,
           l1_W, l1_b, l1_Wih, l1_bih, l1_Whh, l1_bhh,
           l2_W, l2_b, l2_Wih, l2_bih, l2_Whh, l2_bhh,
           l3_W, l3_b, l3_Wih, l3_bih, l3_Whh, l3_bhh,
           fc1_W, fc1_b, fc2_W, fc2_b):
    src = edge_index[0]
    dst = edge_index[1]
    pad = PADE - E
    srcp = jnp.concatenate([src, jnp.zeros((pad,), jnp.int32)])
    pcore = []
    for c in range(NC):
        local = dst - c * HALF
        local = jnp.where((local >= 0) & (local < HALF), local, HALF)
        localp = jnp.concatenate([local, jnp.full((pad,), HALF, jnp.int32)])
        pcore.append(srcp | (localp << 16))
    pkp = jnp.stack(pcore).reshape(NC, NS, NCHUNK, CHUNK)

    layers = [
        (l0_W, l0_b, l0_Wih, l0_bih, l0_Whh, l0_bhh),
        (l1_W, l1_b, l1_Wih, l1_bih, l1_Whh, l1_bhh),
        (l2_W, l2_b, l2_Wih, l2_bih, l2_Whh, l2_bhh),
        (l3_W, l3_b, l3_Wih, l3_bih, l3_Whh, l3_bhh),
    ]
    x = h
    hW = _mm(x, l0_W.T, l0_b)
    for l, (W, b, Wih, bih, Whh, bhh) in enumerate(layers):
        part = _sc_scatter(hW, pkp)
        if l < 3:
            Wn, bn = layers[l + 1][0], layers[l + 1][1]
            x, hW = _gru_mm(part, x, Wih.T, bih, Whh.T, bhh, Wn.T, bn)
        else:
            x = _gru(part, x, Wih.T, bih, Whh.T, bhh)
    return _head(x, fc1_W.T, fc1_b, fc2_W.T, fc2_b)


# speedup vs baseline: 1.0055x; 1.0046x over previous
import functools

import jax
import jax.numpy as jnp
from jax import lax
from jax.experimental import pallas as pl
from jax.experimental.pallas import tpu as pltpu
from jax.experimental.pallas import tpu_sc as plsc

N = 10000
E = 320000
H = 128
C = 40

NC = 2
NS = 16
LANES = 16

CHUNK = 128
NBUF = 4
SDEPTH = 2
LOOKAHEAD = NBUF - SDEPTH
NCHUNK = 157
NLOOP = -(-NCHUNK // NBUF)
PADE = NS * NCHUNK * CHUNK
HALF = 5120
ACC_ROWS = 5232
ZPT = ACC_ROWS // NS
WPT = HALF // NS
OUT_N = NC * HALF

BLK = 1000


def _sc_scatter(hW, pk_idx):
    mesh = plsc.VectorSubcoreMesh(core_axis_name="c", subcore_axis_name="s")

    @functools.partial(
        pl.kernel,
        out_type=jax.ShapeDtypeStruct((OUT_N, H), jnp.float32),
        mesh=mesh,
        scratch_types=[
            pltpu.VMEM((NCHUNK, CHUNK), jnp.int32),
            pltpu.VMEM((NBUF, CHUNK), jnp.int32),
            pltpu.VMEM((NBUF, CHUNK), jnp.int32),
            [pltpu.VMEM((CHUNK, H), jnp.float32) for _ in range(NBUF)],
            pltpu.VMEM_SHARED((ACC_ROWS, H), jnp.float32),
            [pltpu.SemaphoreType.DMA for _ in range(NBUF)],
            [pltpu.SemaphoreType.DMA for _ in range(NBUF)],
        ],
    )
    def k(hW_hbm, pk_hbm, out_hbm, pk_v, src_r, dst_r, bufs, acc, gsems, ssems):
        c = lax.axis_index("c")
        s = lax.axis_index("s")
        buf0 = bufs[0]

        def unpack(ch, slot):
            for kk in range(CHUNK // LANES):
                v = pk_v[ch, pl.ds(kk * LANES, LANES)]
                src_r[slot, pl.ds(kk * LANES, LANES)] = v & 0xFFFF
                dst_r[slot, pl.ds(kk * LANES, LANES)] = lax.shift_right_logical(v, 16)

        def zrow(r, carry):
            for kk in range(H // LANES):
                buf0[r, pl.ds(kk * LANES, LANES)] = jnp.zeros((LANES,), jnp.float32)
            return carry

        lax.fori_loop(0, CHUNK, zrow, 0)
        zbase = s * ZPT
        done = 0
        while done < ZPT:
            n = min(CHUNK, ZPT - done)
            pltpu.sync_copy(buf0.at[pl.ds(0, n)], acc.at[pl.ds(zbase + done, n)])
            done += n

        pltpu.sync_copy(pk_hbm.at[c, s], pk_v)
        for ch in range(LOOKAHEAD):
            unpack(ch, ch)
            pltpu.async_copy(hW_hbm.at[src_r.at[ch]], bufs[ch], gsems[ch])

        plsc.subcore_barrier()

        def step(i, carry):
            for b in range(NBUF):
                ch = NBUF * i + b

                @pl.when(ch < NCHUNK)
                def _(b=b, ch=ch):
                    buf = bufs[b]
                    pltpu.make_async_copy(hW_hbm.at[src_r.at[b]], buf, gsems[b]).wait()
                    pltpu.async_copy(buf, acc.at[dst_r.at[b]], ssems[b], add=True)

                    nxt = ch + LOOKAHEAD
                    nb = (b + LOOKAHEAD) % NBUF

                    @pl.when(nxt < NCHUNK)
                    def _():
                        @pl.when(ch >= SDEPTH)
                        def _():
                            pltpu.make_async_copy(
                                hW_hbm.at[src_r.at[0]], bufs[nb], ssems[nb]).wait()

                        unpack(nxt, nb)
                        pltpu.async_copy(hW_hbm.at[src_r.at[nb]], bufs[nb], gsems[nb])
            return carry

        lax.fori_loop(0, NLOOP, step, 0)
        for b in range(NBUF):
            pltpu.make_async_copy(hW_hbm.at[src_r.at[0]], bufs[b], ssems[b]).wait()

        plsc.subcore_barrier()
        lbase = s * WPT
        obase = c * HALF + lbase
        off = 0
        while off < WPT:
            nrows = min(CHUNK, WPT - off)
            pltpu.sync_copy(acc.at[pl.ds(lbase + off, nrows)], buf0.at[pl.ds(0, nrows)])
            pltpu.sync_copy(buf0.at[pl.ds(0, nrows)], out_hbm.at[pl.ds(obase + off, nrows)])
            off += nrows

    return k(hW, pk_idx)


def _mm_kernel(x_ref, w_ref, b_ref, o_ref):
    o_ref[...] = (
        jnp.dot(x_ref[...], w_ref[...], preferred_element_type=jnp.float32) + b_ref[...]
    )


def _mm(x, WT, b):
    K = WT.shape[1]
    return pl.pallas_call(
        _mm_kernel,
        grid=(N // BLK,),
        in_specs=[
            pl.BlockSpec((BLK, H), lambda i: (i, 0)),
            pl.BlockSpec((H, K), lambda i: (0, 0)),
            pl.BlockSpec((1, K), lambda i: (0, 0)),
        ],
        out_specs=pl.BlockSpec((BLK, K), lambda i: (i, 0)),
        out_shape=jax.ShapeDtypeStruct((N, K), jnp.float32),
    )(x, WT, b.reshape(1, K))


def _gru_core(a, h, wih, bih, whh, bhh):
    gi = jnp.dot(a, wih, preferred_element_type=jnp.float32) + bih
    gh = jnp.dot(h, whh, preferred_element_type=jnp.float32) + bhh
    r = jax.nn.sigmoid(gi[:, :H] + gh[:, :H])
    z = jax.nn.sigmoid(gi[:, H : 2 * H] + gh[:, H : 2 * H])
    n = jnp.tanh(gi[:, 2 * H :] + r * gh[:, 2 * H :])
    return jnp.maximum((1.0 - z) * n + z * h, 0.0)


def _gru_mm_kernel(a_ref, h_ref, wih_ref, bih_ref, whh_ref, bhh_ref,
                   wn_ref, bn_ref, o_ref, m_ref):
    hn = _gru_core(a_ref[...], h_ref[...], wih_ref[...], bih_ref[...],
                   whh_ref[...], bhh_ref[...])
    o_ref[...] = hn
    m_ref[...] = jnp.dot(hn, wn_ref[...], preferred_element_type=jnp.float32) + bn_ref[...]


def _gru_mm(part, h, WihT, bih, WhhT, bhh, WnT, bn):
    return pl.pallas_call(
        _gru_mm_kernel,
        grid=(N // BLK,),
        in_specs=[
            pl.BlockSpec((BLK, H), lambda i: (i, 0)),
            pl.BlockSpec((BLK, H), lambda i: (i, 0)),
            pl.BlockSpec((H, 3 * H), lambda i: (0, 0)),
            pl.BlockSpec((1, 3 * H), lambda i: (0, 0)),
            pl.BlockSpec((H, 3 * H), lambda i: (0, 0)),
            pl.BlockSpec((1, 3 * H), lambda i: (0, 0)),
            pl.BlockSpec((H, H), lambda i: (0, 0)),
            pl.BlockSpec((1, H), lambda i: (0, 0)),
        ],
        out_specs=[
            pl.BlockSpec((BLK, H), lambda i: (i, 0)),
            pl.BlockSpec((BLK, H), lambda i: (i, 0)),
        ],
        out_shape=[
            jax.ShapeDtypeStruct((N, H), jnp.float32),
            jax.ShapeDtypeStruct((N, H), jnp.float32),
        ],
    )(part, h, WihT, bih.reshape(1, 3 * H), WhhT, bhh.reshape(1, 3 * H),
      WnT, bn.reshape(1, H))


def _gru_head_kernel(a_ref, h_ref, wih_ref, bih_ref, whh_ref, bhh_ref,
                     w1_ref, b1_ref, w2_ref, b2_ref, o_ref, acc_ref):
    i = pl.program_id(0)
    hn = _gru_core(a_ref[...], h_ref[...], wih_ref[...], bih_ref[...],
                   whh_ref[...], bhh_ref[...])

    @pl.when(i == 0)
    def _():
        acc_ref[...] = jnp.zeros_like(acc_ref)

    acc_ref[...] += jnp.sum(hn.reshape(BLK // 8, 8, H), axis=0)

    @pl.when(i == N // BLK - 1)
    def _():
        m = jnp.sum(acc_ref[...], axis=0, keepdims=True) * (1.0 / N)
        o1 = jnp.dot(m, w1_ref[...], preferred_element_type=jnp.float32) + b1_ref[...]
        o1 = jnp.where(o1 > 0.0, o1, jnp.exp(o1) - 1.0)
        o2 = jnp.dot(o1, w2_ref[...], preferred_element_type=jnp.float32) + b2_ref[...]
        mx = jnp.max(o2, axis=0, keepdims=True)
        lse = mx + jnp.log(jnp.sum(jnp.exp(o2 - mx), axis=0, keepdims=True))
        o_ref[...] = o2 - lse


def _gru_head(part, h, WihT, bih, WhhT, bhh, fc1_WT, fc1_b, fc2_WT, fc2_b):
    return pl.pallas_call(
        _gru_head_kernel,
        grid=(N // BLK,),
        in_specs=[
            pl.BlockSpec((BLK, H), lambda i: (i, 0)),
            pl.BlockSpec((BLK, H), lambda i: (i, 0)),
            pl.BlockSpec((H, 3 * H), lambda i: (0, 0)),
            pl.BlockSpec((1, 3 * H), lambda i: (0, 0)),
            pl.BlockSpec((H, 3 * H), lambda i: (0, 0)),
            pl.BlockSpec((1, 3 * H), lambda i: (0, 0)),
            pl.BlockSpec((H, H), lambda i: (0, 0)),
            pl.BlockSpec((1, H), lambda i: (0, 0)),
            pl.BlockSpec((H, C), lambda i: (0, 0)),
            pl.BlockSpec((1, C), lambda i: (0, 0)),
        ],
        out_specs=pl.BlockSpec((1, C), lambda i: (0, 0)),
        out_shape=jax.ShapeDtypeStruct((1, C), jnp.float32),
        scratch_shapes=[pltpu.VMEM((8, H), jnp.float32)],
    )(part, h, WihT, bih.reshape(1, 3 * H), WhhT, bhh.reshape(1, 3 * H),
      fc1_WT, fc1_b.reshape(1, H), fc2_WT, fc2_b.reshape(1, C))


def kernel(h, edge_index, e, l0_W, l0_b, l0_Wih, l0_bih, l0_Whh, l0_bhh,
           l1_W, l1_b, l1_Wih, l1_bih, l1_Whh, l1_bhh,
           l2_W, l2_b, l2_Wih, l2_bih, l2_Whh, l2_bhh,
           l3_W, l3_b, l3_Wih, l3_bih, l3_Whh, l3_bhh,
           fc1_W, fc1_b, fc2_W, fc2_b):
    src = edge_index[0]
    dst = edge_index[1]
    pad = PADE - E
    srcp = jnp.concatenate([src, jnp.zeros((pad,), jnp.int32)])
    pcore = []
    for c in range(NC):
        local = dst - c * HALF
        local = jnp.where((local >= 0) & (local < HALF), local, HALF)
        localp = jnp.concatenate([local, jnp.full((pad,), HALF, jnp.int32)])
        pcore.append(srcp | (localp << 16))
    pkp = jnp.stack(pcore).reshape(NC, NS, NCHUNK, CHUNK)

    layers = [
        (l0_W, l0_b, l0_Wih, l0_bih, l0_Whh, l0_bhh),
        (l1_W, l1_b, l1_Wih, l1_bih, l1_Whh, l1_bhh),
        (l2_W, l2_b, l2_Wih, l2_bih, l2_Whh, l2_bhh),
        (l3_W, l3_b, l3_Wih, l3_bih, l3_Whh, l3_bhh),
    ]
    x = h
    hW = _mm(x, l0_W.T, l0_b)
    for l, (W, b, Wih, bih, Whh, bhh) in enumerate(layers):
        part = _sc_scatter(hW, pkp)
        if l < 3:
            Wn, bn = layers[l + 1][0], layers[l + 1][1]
            x, hW = _gru_mm(part, x, Wih.T, bih, Whh.T, bhh, Wn.T, bn)
        else:
            out = _gru_head(part, x, Wih.T, bih, Whh.T, bhh,
                            fc1_W.T, fc1_b, fc2_W.T, fc2_b)
    return out
